# Initial kernel scaffold; baseline (speedup 1.0000x reference)
import jax
import jax.numpy as jnp
from jax.experimental import pallas as pl


def kernel(x, edge_index, drug_index, W1, b1, W2, b2, W3, b3, g1, be1, g2, be2, g3, be3, P1, P2):
    raise NotImplementedError("write your pallas kernel here")


# speedup vs baseline: 5.3076x; 5.3076x over previous
import functools

import jax
import jax.numpy as jnp
from jax import lax
from jax.experimental import pallas as pl
from jax.experimental.pallas import tpu as pltpu
from jax.experimental.pallas import tpu_sc as plsc

N = 10000
E = 160000
EPS = 1e-5
LANE = 128
NSUB = 16
NCORE = 2
EB = 128
NJ = 80
E_PAD = NSUB * NJ * EB
NJD = E_PAD // (NCORE * NSUB * EB)
ROWS_T = 640
ACC_R = NSUB * ROWS_T
LAST_R = N - (NSUB - 1) * ROWS_T
RB = 1000
NRB = N // RB


def _seg_mesh():
  return plsc.VectorSubcoreMesh(core_axis_name="c", subcore_axis_name="s")


def _make_seg(nchunk):
  cpc = nchunk // NCORE
  IB = 16

  @functools.partial(
      pl.kernel,
      out_type=jax.ShapeDtypeStruct((nchunk, N, LANE), jnp.float32),
      mesh=_seg_mesh(),
      scratch_types=[
          pltpu.VMEM((IB, EB), jnp.int32),
          pltpu.VMEM((IB, EB), jnp.int32),
          pltpu.VMEM((EB, LANE), jnp.float32),
          pltpu.VMEM((64, LANE), jnp.float32),
          pltpu.VMEM_SHARED((ACC_R, LANE), jnp.float32),
          pltpu.SemaphoreType.DMA,
      ],
  )
  def seg(s_hbm, ridx_hbm, cidx_hbm, zeros_hbm, out_hbm,
          idx_r, idx_c, rows, zbuf, acc, sem):
    c = lax.axis_index("c")
    s = lax.axis_index("s")
    base = pl.multiple_of(s * ROWS_T, ROWS_T)
    pltpu.sync_copy(zeros_hbm, zbuf)
    for k in range(cpc):
      chunk = c * cpc + k
      for m in range(ROWS_T // 64):
        pltpu.sync_copy(zbuf, acc.at[pl.ds(base + m * 64, 64)])
      plsc.subcore_barrier()

      for ib in range(NJ // IB):
        pltpu.sync_copy(ridx_hbm.at[chunk, s, pl.ds(ib * IB, IB)], idx_r)
        pltpu.sync_copy(cidx_hbm.at[s, pl.ds(ib * IB, IB)], idx_c)

        def body(j, carry):
          pltpu.async_copy(s_hbm.at[idx_r.at[j]], rows, sem).wait()
          pltpu.sync_copy(rows, acc.at[idx_c.at[j]], add=True)
          return carry

        lax.fori_loop(0, IB, body, 0)
      plsc.subcore_barrier()

      @pl.when(s < NSUB - 1)
      def _():
        pltpu.sync_copy(acc.at[pl.ds(base, ROWS_T)],
                        out_hbm.at[chunk, pl.ds(base, ROWS_T)])

      @pl.when(s == NSUB - 1)
      def _():
        pltpu.sync_copy(acc.at[pl.ds((NSUB - 1) * ROWS_T, LAST_R)],
                        out_hbm.at[chunk, pl.ds((NSUB - 1) * ROWS_T, LAST_R)])

  return seg


def _make_deg():
  @functools.partial(
      pl.kernel,
      out_type=jax.ShapeDtypeStruct((NCORE, N, 16), jnp.float32),
      mesh=_seg_mesh(),
      scratch_types=[
          pltpu.VMEM((NJD, EB), jnp.int32),
          pltpu.VMEM((EB, 16), jnp.float32),
          pltpu.VMEM((EB, 16), jnp.float32),
          pltpu.VMEM_SHARED((ACC_R, 16), jnp.float32),
          pltpu.SemaphoreType.DMA,
      ],
  )
  def deg(cidx_hbm, ones_hbm, zeros_hbm, out_hbm,
          idx_c, ones_v, zbuf, acc, sem):
    c = lax.axis_index("c")
    s = lax.axis_index("s")
    base = pl.multiple_of(s * ROWS_T, ROWS_T)
    pltpu.sync_copy(cidx_hbm.at[c, s], idx_c)
    pltpu.sync_copy(ones_hbm, ones_v)
    pltpu.sync_copy(zeros_hbm, zbuf)
    for m in range(ROWS_T // EB):
      pltpu.sync_copy(zbuf, acc.at[pl.ds(base + m * EB, EB)])
    plsc.subcore_barrier()

    def body(j, carry):
      pltpu.sync_copy(ones_v, acc.at[idx_c.at[j]], add=True)
      return carry

    lax.fori_loop(0, NJD, body, 0)
    plsc.subcore_barrier()

    @pl.when(s < NSUB - 1)
    def _():
      pltpu.sync_copy(acc.at[pl.ds(base, ROWS_T)],
                      out_hbm.at[c, pl.ds(base, ROWS_T)])

    @pl.when(s == NSUB - 1)
    def _():
      pltpu.sync_copy(acc.at[pl.ds((NSUB - 1) * ROWS_T, LAST_R)],
                      out_hbm.at[c, pl.ds((NSUB - 1) * ROWS_T, LAST_R)])

  return deg


def _make_pair_gather():
  npair = 2048
  per_tile = npair // (NCORE * NSUB)

  nw = NCORE * NSUB

  @functools.partial(
      pl.kernel,
      out_type=jax.ShapeDtypeStruct((2, npair, LANE), jnp.float32),
      mesh=_seg_mesh(),
      scratch_types=[
          pltpu.VMEM((per_tile,), jnp.int32),
          pltpu.VMEM((per_tile, LANE), jnp.float32),
          pltpu.SemaphoreType.DMA,
      ],
  )
  def pg(y3_hbm, gidx_hbm, out_hbm, idxv, buf, sem):
    c = lax.axis_index("c")
    s = lax.axis_index("s")
    wid = s * NCORE + c
    base = pl.multiple_of(wid * per_tile, per_tile)
    for k in range(2):
      pltpu.sync_copy(gidx_hbm.at[k, wid], idxv)
      pltpu.async_copy(y3_hbm.at[idxv], buf, sem).wait()
      pltpu.sync_copy(buf, out_hbm.at[k, pl.ds(base, per_tile)])

  return pg


def _lrelu(x):
  return jnp.where(x > 0, x, 0.1 * x)


def _k0_body(x_ref, w1t_ref, b1_ref, degp_ref, s1_ref, dis_ref):
  p0 = degp_ref[0, :, 0:1]
  p1 = degp_ref[1, :, 0:1]
  dis = lax.rsqrt(1.0 + p0 + p1)
  dis_ref[...] = dis
  nco = s1_ref.shape[0]
  for co in range(nco):
    h = jnp.dot(x_ref[...], w1t_ref[:, co * LANE:(co + 1) * LANE],
                preferred_element_type=jnp.float32)
    s1_ref[co] = dis * (h + b1_ref[0:1, co * LANE:(co + 1) * LANE])


def _tc_front(x, w1t, b1, degp, nch_out):
  return pl.pallas_call(
      _k0_body,
      grid=(NRB,),
      in_specs=[
          pl.BlockSpec((RB, x.shape[1]), lambda i: (i, 0)),
          pl.BlockSpec(w1t.shape, lambda i: (0, 0)),
          pl.BlockSpec(b1.shape, lambda i: (0, 0)),
          pl.BlockSpec((NCORE, RB, 16), lambda i: (0, i, 0)),
      ],
      out_specs=[
          pl.BlockSpec((nch_out, RB, LANE), lambda i: (0, i, 0)),
          pl.BlockSpec((RB, 1), lambda i: (i, 0)),
      ],
      out_shape=[
          jax.ShapeDtypeStruct((nch_out, N, LANE), jnp.float32),
          jax.ShapeDtypeStruct((N, 1), jnp.float32),
      ],
  )(x, w1t, b1, degp)


def _k1_body(p_ref, s_ref, dis_ref, y_ref, stats_ref, acc):
  i = pl.program_id(0)
  dis = dis_ref[...][None]
  y = dis * (p_ref[...] + s_ref[...])
  y_ref[...] = y

  @pl.when(i == 0)
  def _():
    acc[...] = jnp.zeros_like(acc)

  acc[0] += jnp.sum(y, axis=1)
  acc[1] += jnp.sum(y * y, axis=1)

  @pl.when(i == NRB - 1)
  def _():
    stats_ref[...] = acc[...]


def _tc_stats(p, s, dis):
  nch = p.shape[0]
  return pl.pallas_call(
      _k1_body,
      grid=(NRB,),
      in_specs=[
          pl.BlockSpec((nch, RB, LANE), lambda i: (0, i, 0)),
          pl.BlockSpec((nch, RB, LANE), lambda i: (0, i, 0)),
          pl.BlockSpec((RB, 1), lambda i: (i, 0)),
      ],
      out_specs=[
          pl.BlockSpec((nch, RB, LANE), lambda i: (0, i, 0)),
          pl.BlockSpec((2, nch, LANE), lambda i: (0, 0, 0)),
      ],
      out_shape=[
          jax.ShapeDtypeStruct((nch, N, LANE), jnp.float32),
          jax.ShapeDtypeStruct((2, nch, LANE), jnp.float32),
      ],
      scratch_shapes=[pltpu.VMEM((2, nch, LANE), jnp.float32)],
  )(p, s, dis)


def _k2_body(y_ref, stats_ref, g_ref, b_ref, wt_ref, bias_ref, dis_ref,
             out_ref):
  nch_in = y_ref.shape[0]
  nch_out = out_ref.shape[0]
  mu = stats_ref[0] * (1.0 / N)
  var = stats_ref[1] * (1.0 / N) - mu * mu
  scale = lax.rsqrt(var + EPS) * g_ref[...]
  shift = b_ref[...] - mu * scale
  dis = dis_ref[...]
  zs = []
  for ci in range(nch_in):
    z = y_ref[ci] * scale[ci][None] + shift[ci][None]
    zs.append(_lrelu(z))
  for co in range(nch_out):
    acc = jnp.zeros((y_ref.shape[1], LANE), jnp.float32)
    for ci in range(nch_in):
      acc += jnp.dot(zs[ci],
                     wt_ref[ci * LANE:(ci + 1) * LANE, co * LANE:(co + 1) * LANE],
                     preferred_element_type=jnp.float32)
    out_ref[co] = dis * (acc + bias_ref[0:1, co * LANE:(co + 1) * LANE])


def _tc_bn_matmul(y, stats, g, b, wt, bias, dis, nch_out):
  nch_in = y.shape[0]
  return pl.pallas_call(
      _k2_body,
      grid=(NRB,),
      in_specs=[
          pl.BlockSpec((nch_in, RB, LANE), lambda i: (0, i, 0)),
          pl.BlockSpec((2, nch_in, LANE), lambda i: (0, 0, 0)),
          pl.BlockSpec((nch_in, LANE), lambda i: (0, 0)),
          pl.BlockSpec((nch_in, LANE), lambda i: (0, 0)),
          pl.BlockSpec(wt.shape, lambda i: (0, 0)),
          pl.BlockSpec(bias.shape, lambda i: (0, 0)),
          pl.BlockSpec((RB, 1), lambda i: (i, 0)),
      ],
      out_specs=pl.BlockSpec((nch_out, RB, LANE), lambda i: (0, i, 0)),
      out_shape=jax.ShapeDtypeStruct((nch_out, N, LANE), jnp.float32),
  )(y, stats, g, b, wt, bias, dis)


def _k6_body(pairs_ref, stats_ref, g_ref, b_ref, p1_ref, p2_ref, p1t_ref,
             out_ref):
  mu = stats_ref[0] * (1.0 / N)
  var = stats_ref[1] * (1.0 / N) - mu * mu
  scale = lax.rsqrt(var + EPS) * g_ref[...]
  shift = b_ref[...] - mu * scale
  z0 = _lrelu(pairs_ref[0] * scale[0][None] + shift[0][None])
  z1 = _lrelu(pairs_ref[1] * scale[1][None] + shift[1][None])
  a = jnp.concatenate([z0[:1024], z1[:1024]], axis=1)
  bb = jnp.concatenate([z0[1024:], z1[1024:]], axis=1)
  m = jnp.dot(jnp.dot(p1_ref[...], p2_ref[...],
                      preferred_element_type=jnp.float32),
              p1t_ref[...], preferred_element_type=jnp.float32)
  t = jnp.dot(a, m, preferred_element_type=jnp.float32)
  out_ref[...] = jnp.sum(t * bb, axis=1, keepdims=True)


def _tc_decoder(pairs, stats, g, b, p1, p2, p1t):
  return pl.pallas_call(
      _k6_body,
      grid=(1,),
      in_specs=[
          pl.BlockSpec(pairs.shape, lambda i: (0, 0, 0)),
          pl.BlockSpec(stats.shape, lambda i: (0, 0, 0)),
          pl.BlockSpec(g.shape, lambda i: (0, 0)),
          pl.BlockSpec(b.shape, lambda i: (0, 0)),
          pl.BlockSpec(p1.shape, lambda i: (0, 0)),
          pl.BlockSpec(p2.shape, lambda i: (0, 0)),
          pl.BlockSpec(p1t.shape, lambda i: (0, 0)),
      ],
      out_specs=pl.BlockSpec((1024, 1), lambda i: (0, 0)),
      out_shape=jax.ShapeDtypeStruct((1024, 1), jnp.float32),
  )(pairs, stats, g, b, p1, p2, p1t)


def kernel(x, edge_index, drug_index, W1, b1, W2, b2, W3, b3,
           g1, be1, g2, be2, g3, be3, P1, P2):
  f32 = jnp.float32
  row = edge_index[0]
  col = edge_index[1]
  npad = E_PAD - E
  rowp = jnp.concatenate([row, jnp.zeros((npad,), jnp.int32)])
  colp = jnp.concatenate([col, jnp.full((npad,), N, jnp.int32)])
  row_t = rowp.reshape(NSUB, NJ, EB)
  cidx_seg = colp.reshape(NSUB, NJ, EB)
  cidx_deg = colp.reshape(NCORE, NSUB, NJD, EB)
  ridx4 = (row_t[None] + (jnp.arange(4, dtype=jnp.int32) * N)[:, None, None, None])
  ridx2 = ridx4[:2]

  zeros128 = jnp.zeros((64, LANE), f32)
  zeros16 = jnp.zeros((EB, 16), f32)
  ones16 = jnp.ones((EB, 16), f32)

  di = drug_index.reshape(-1, 2)
  gidx0 = jnp.concatenate([(di[:, 0] - 1) % N, (di[:, 1] - 1) % N])
  gidx = jnp.stack([gidx0, gidx0 + N]).astype(jnp.int32).reshape(2, NCORE * NSUB, -1)

  w1t = W1.T
  w2t = W2.T
  w3t = W3.T
  b1r = b1.reshape(1, -1)
  b2r = b2.reshape(1, -1)
  b3r = b3.reshape(1, -1)
  g1r = g1.reshape(4, LANE)
  be1r = be1.reshape(4, LANE)
  g2r = g2.reshape(4, LANE)
  be2r = be2.reshape(4, LANE)
  g3r = g3.reshape(2, LANE)
  be3r = be3.reshape(2, LANE)

  seg4 = _make_seg(4)
  seg2 = _make_seg(2)
  deg_k = _make_deg()
  pg_k = _make_pair_gather()

  degp = deg_k(cidx_deg, ones16, zeros16)

  s1, dis = _tc_front(x, w1t, b1r, degp, 4)
  p1sum = seg4(s1.reshape(4 * N, LANE), ridx4, cidx_seg, zeros128)
  y1, st1 = _tc_stats(p1sum, s1, dis)
  s2 = _tc_bn_matmul(y1, st1, g1r, be1r, w2t, b2r, dis, 4)
  p2sum = seg4(s2.reshape(4 * N, LANE), ridx4, cidx_seg, zeros128)
  y2, st2 = _tc_stats(p2sum, s2, dis)
  s3 = _tc_bn_matmul(y2, st2, g2r, be2r, w3t, b3r, dis, 2)
  p3sum = seg2(s3.reshape(2 * N, LANE), ridx2, cidx_seg, zeros128)
  y3, st3 = _tc_stats(p3sum, s3, dis)
  pairs = pg_k(y3.reshape(2 * N, LANE), gidx)
  ypred = _tc_decoder(pairs, st3, g3r, be3r, P1, P2, P1.T)
  return ypred
